# baseline (speedup 1.0000x reference)
import functools

import jax
import jax.numpy as jnp
from jax import lax
from jax.experimental import pallas as pl
from jax.experimental.pallas import tpu as pltpu
from jax.experimental.pallas import tpu_sc as plsc

CH = 128
LANES = 16
NW = 32
CPW = 80
NBUF = 2


@functools.lru_cache(maxsize=None)
def _make_sc_aggregate(n_nodes: int, d: int):
    rows_per_tile = (n_nodes // (16 * 8)) * 8
    hop = 104
    n_hops = rows_per_tile // hop
    assert n_hops * hop == rows_per_tile
    tail = n_nodes - 16 * rows_per_tile
    assert 0 <= tail <= CH and tail % 8 == 0
    vregs_per_row = d // LANES

    mesh = plsc.VectorSubcoreMesh(core_axis_name="c", subcore_axis_name="s")

    rows_t = pltpu.VMEM((CH, d), jnp.float32)
    idx_t = pltpu.VMEM((CH,), jnp.int32)
    w_t = pltpu.VMEM((CH,), jnp.float32)
    sem_t = pltpu.SemaphoreType.DMA

    @functools.partial(
        pl.kernel,
        mesh=mesh,
        out_type=jax.ShapeDtypeStruct((2, n_nodes, d), jnp.float32),
        scratch_types=(
            [rows_t] * NBUF + [idx_t] * NBUF + [idx_t] * NBUF + [w_t] * NBUF
            + [pltpu.VMEM_SHARED((n_nodes, d), jnp.float32)]
            + [sem_t] * NBUF + [sem_t] * NBUF + [sem_t] * NBUF
        ),
    )
    def sc_agg(row_hbm, col_hbm, w_hbm, x_hbm, out_hbm, *scr):
        rows = scr[0:NBUF]
        col = scr[NBUF:2 * NBUF]
        dst = scr[2 * NBUF:3 * NBUF]
        wgt = scr[3 * NBUF:4 * NBUF]
        accum = scr[4 * NBUF]
        sem_g = scr[4 * NBUF + 1:4 * NBUF + 1 + NBUF]

        c = lax.axis_index("c")
        s = lax.axis_index("s")
        wid = s * 2 + c
        base = wid * CPW

        zero16 = jnp.zeros((LANES,), jnp.float32)

        def _zero_row(i, _):
            for j in range(vregs_per_row):
                rows[0][i, pl.ds(j * LANES, LANES)] = zero16
            return 0

        lax.fori_loop(0, CH, _zero_row, 0)
        for h in range(n_hops):
            pltpu.sync_copy(rows[0].at[pl.ds(0, hop)],
                            accum.at[pl.ds(s * rows_per_tile + h * hop, hop)])
        if tail:
            @pl.when(s == 15)
            def _():
                pltpu.sync_copy(rows[0].at[pl.ds(0, tail)],
                                accum.at[pl.ds(16 * rows_per_tile, tail)])
        plsc.subcore_barrier()

        def _idx_load(k, b):
            off = (k * NW + wid) * CH
            pltpu.sync_copy(col_hbm.at[pl.ds(off, CH)], col[b])
            pltpu.sync_copy(row_hbm.at[pl.ds(off, CH)], dst[b])
            pltpu.sync_copy(w_hbm.at[pl.ds(off, CH)], wgt[b])

        def _gather_start(b):
            pltpu.async_copy(x_hbm.at[col[b]], rows[b], sem_g[b])

        def _gather_wait(b):
            pltpu.make_async_copy(x_hbm.at[col[b]], rows[b], sem_g[b]).wait()

        def _scatter(b):
            pltpu.sync_copy(rows[b], accum.at[dst[b]], add=True)

        def _scale(b):
            def _group(g, _):
                w16 = wgt[b][pl.ds(g * LANES, LANES)]
                for lane in range(LANES):
                    e = g * LANES + lane
                    wvec = jnp.full((LANES,), w16[lane], jnp.float32)
                    for j in range(vregs_per_row):
                        rows[b][e, pl.ds(j * LANES, LANES)] = (
                            rows[b][e, pl.ds(j * LANES, LANES)] * wvec)
                return 0

            lax.fori_loop(0, CH // LANES, _group, 0)

        _idx_load(0, 0)
        _gather_start(0)

        def _step(k, b):
            bn = (b + 1) % NBUF

            @pl.when(k + 1 < CPW)
            def _():
                _idx_load(k + 1, bn)
                _gather_start(bn)

            _gather_wait(b)
            _scale(b)
            _scatter(b)

        def _body(g, _):
            for bb in range(NBUF):
                _step(g * NBUF + bb, bb)
            return 0

        lax.fori_loop(0, CPW // NBUF, _body, 0)
        plsc.subcore_barrier()

        for h in range(n_hops):
            r0 = s * rows_per_tile + h * hop
            pltpu.sync_copy(accum.at[pl.ds(r0, hop)], rows[0].at[pl.ds(0, hop)])
            pltpu.sync_copy(rows[0].at[pl.ds(0, hop)], out_hbm.at[c, pl.ds(r0, hop)])
        if tail:
            @pl.when(s == 15)
            def _():
                r0 = 16 * rows_per_tile
                pltpu.sync_copy(accum.at[pl.ds(r0, tail)], rows[0].at[pl.ds(0, tail)])
                pltpu.sync_copy(rows[0].at[pl.ds(0, tail)],
                                out_hbm.at[c, pl.ds(r0, tail)])

    return sc_agg


def _tc_body(x_ref, a_ref, ws_ref, wn_ref, b_ref, o_ref):
    xb = x_ref[...]
    ab = a_ref[0] + a_ref[1]
    dn = (((1,), (1,)), ((), ()))
    o_ref[...] = (
        lax.dot_general(xb, ws_ref[...], dn, preferred_element_type=jnp.float32)
        + lax.dot_general(ab, wn_ref[...], dn, preferred_element_type=jnp.float32)
        + b_ref[...]
    )


@functools.lru_cache(maxsize=None)
def _make_tc_linear(n_nodes: int, d: int):
    br = 1000
    assert n_nodes % br == 0
    grid = (n_nodes // br,)
    return pl.pallas_call(
        _tc_body,
        grid=grid,
        in_specs=[
            pl.BlockSpec((br, d), lambda i: (i, 0)),
            pl.BlockSpec((2, br, d), lambda i: (0, i, 0)),
            pl.BlockSpec((d, d), lambda i: (0, 0)),
            pl.BlockSpec((d, d), lambda i: (0, 0)),
            pl.BlockSpec((1, d), lambda i: (0, 0)),
        ],
        out_specs=pl.BlockSpec((br, d), lambda i: (i, 0)),
        out_shape=jax.ShapeDtypeStruct((n_nodes, d), jnp.float32),
    )


def kernel(x, edge_index, edge_weight, num_nodes, W_self, b_self, W_neigh, b_neigh):
    n, d = x.shape
    e = edge_index.shape[1]
    ei = edge_index.astype(jnp.int32)
    row = (ei[0] % jnp.asarray(num_nodes, jnp.int32)).astype(jnp.int32)
    col = ei[1]
    ep = NW * CPW * CH
    pad = ep - e
    assert pad >= 0
    row1 = jnp.concatenate([row, jnp.zeros((pad,), jnp.int32)])
    col1 = jnp.concatenate([col, jnp.zeros((pad,), jnp.int32)])
    w1 = jnp.concatenate(
        [edge_weight.astype(jnp.float32), jnp.zeros((pad,), jnp.float32)])
    agg = _make_sc_aggregate(n, d)(row1, col1, w1, x)
    bias = (b_self + b_neigh).reshape(1, d).astype(jnp.float32)
    return _make_tc_linear(n, d)(x, agg, W_self, W_neigh, bias)


# speedup vs baseline: 2.0313x; 2.0313x over previous
import functools

import jax
import jax.numpy as jnp
from jax import lax
from jax.experimental import pallas as pl
from jax.experimental.pallas import tpu as pltpu
from jax.experimental.pallas import tpu_sc as plsc

CH = 128
LANES = 16
NW = 32


@functools.lru_cache(maxsize=None)
def _make_sc_aggregate(n_nodes: int, d: int, n_edges: int):
    assert n_edges % CH == 0
    n_chunks = n_edges // CH
    per = n_chunks // NW
    per -= per % 2
    extra = n_chunks - per * NW

    rows_per_tile = (n_nodes // (16 * 8)) * 8
    hop = 104
    n_hops = rows_per_tile // hop
    assert n_hops * hop == rows_per_tile
    tail = n_nodes - 16 * rows_per_tile
    assert 0 <= tail <= CH and tail % 8 == 0
    vregs_per_row = d // LANES

    mesh = plsc.VectorSubcoreMesh(core_axis_name="c", subcore_axis_name="s")

    rows_t = pltpu.VMEM((CH, d), jnp.float32)
    idx_t = pltpu.VMEM((CH,), jnp.int32)
    w_t = pltpu.VMEM((CH,), jnp.float32)
    sem_t = pltpu.SemaphoreType.DMA

    @functools.partial(
        pl.kernel,
        mesh=mesh,
        out_type=jax.ShapeDtypeStruct((2, n_nodes, d), jnp.float32),
        scratch_types=[
            rows_t, rows_t, idx_t, idx_t, idx_t, idx_t, w_t, w_t,
            pltpu.VMEM_SHARED((n_nodes, d), jnp.float32),
            sem_t, sem_t, sem_t, sem_t,
        ],
    )
    def sc_agg(row_hbm, col_hbm, w_hbm, x_hbm, out_hbm,
               rows0, rows1, col0, col1, dst0, dst1, wgt0, wgt1,
               accum, semg0, semg1, sems0, sems1):
        rows = (rows0, rows1)
        col = (col0, col1)
        dst = (dst0, dst1)
        wgt = (wgt0, wgt1)
        sem_g = (semg0, semg1)
        sem_s = (sems0, sems1)

        c = lax.axis_index("c")
        s = lax.axis_index("s")
        wid = s * 2 + c

        zero16 = jnp.zeros((LANES,), jnp.float32)

        def _zero_row(i, _):
            for j in range(vregs_per_row):
                rows0[i, pl.ds(j * LANES, LANES)] = zero16
            return 0

        lax.fori_loop(0, CH, _zero_row, 0)
        for h in range(n_hops):
            pltpu.sync_copy(rows0.at[pl.ds(0, hop)],
                            accum.at[pl.ds(s * rows_per_tile + h * hop, hop)])
        if tail:
            @pl.when(s == 15)
            def _():
                pltpu.sync_copy(rows0.at[pl.ds(0, tail)],
                                accum.at[pl.ds(16 * rows_per_tile, tail)])
        plsc.subcore_barrier()

        def _idx_load_colw(k, b):
            off = (k * NW + wid) * CH
            pltpu.sync_copy(col_hbm.at[pl.ds(off, CH)], col[b])
            pltpu.sync_copy(w_hbm.at[pl.ds(off, CH)], wgt[b])

        def _idx_load_dst(k, b):
            off = (k * NW + wid) * CH
            pltpu.sync_copy(row_hbm.at[pl.ds(off, CH)], dst[b])

        def _gather_start(b):
            pltpu.async_copy(x_hbm.at[col[b]], rows[b], sem_g[b])

        def _gather_wait(b):
            pltpu.make_async_copy(x_hbm.at[col[b]], rows[b], sem_g[b]).wait()

        def _scatter_start(b):
            pltpu.async_copy(rows[b], accum.at[dst[b]], sem_s[b], add=True)

        def _scatter_wait(b):
            pltpu.make_async_copy(rows[b], accum.at[dst[b]], sem_s[b]).wait()

        def _scale(b):
            def _group(g, _):
                w16 = wgt[b][pl.ds(g * LANES, LANES)]
                for lane in range(LANES):
                    e = g * LANES + lane
                    wvec = jnp.full((LANES,), w16[lane], jnp.float32)
                    for j in range(vregs_per_row):
                        rows[b][e, pl.ds(j * LANES, LANES)] = (
                            rows[b][e, pl.ds(j * LANES, LANES)] * wvec)
                return 0

            lax.fori_loop(0, CH // LANES, _group, 0)

        def _step(k, b):
            bn = (b + 1) % 2

            @pl.when(k + 1 < per)
            def _():
                _idx_load_colw(k + 1, bn)

            _gather_wait(b)

            @pl.when(k >= 1)
            def _():
                _scatter_wait(bn)

            @pl.when(k + 1 < per)
            def _():
                _gather_start(bn)
                _idx_load_dst(k + 1, bn)

            _scale(b)
            _scatter_start(b)

        _idx_load_colw(0, 0)
        _idx_load_dst(0, 0)
        _gather_start(0)

        def _body(g, _):
            _step(2 * g, 0)
            _step(2 * g + 1, 1)
            return 0

        lax.fori_loop(0, per // 2, _body, 0)
        _scatter_wait((per - 1) % 2)

        if extra:
            @pl.when(wid < extra)
            def _():
                off = (per * NW + wid) * CH
                pltpu.sync_copy(col_hbm.at[pl.ds(off, CH)], col0)
                pltpu.sync_copy(row_hbm.at[pl.ds(off, CH)], dst0)
                pltpu.sync_copy(w_hbm.at[pl.ds(off, CH)], wgt0)
                _gather_start(0)
                _gather_wait(0)
                _scale(0)
                pltpu.sync_copy(rows0, accum.at[dst0], add=True)

        plsc.subcore_barrier()

        for h in range(n_hops):
            r0 = s * rows_per_tile + h * hop
            pltpu.sync_copy(accum.at[pl.ds(r0, hop)], rows0.at[pl.ds(0, hop)])
            pltpu.sync_copy(rows0.at[pl.ds(0, hop)], out_hbm.at[c, pl.ds(r0, hop)])
        if tail:
            @pl.when(s == 15)
            def _():
                r0 = 16 * rows_per_tile
                pltpu.sync_copy(accum.at[pl.ds(r0, tail)], rows0.at[pl.ds(0, tail)])
                pltpu.sync_copy(rows0.at[pl.ds(0, tail)],
                                out_hbm.at[c, pl.ds(r0, tail)])

    return sc_agg


def _tc_body(x_ref, a_ref, ws_ref, wn_ref, b_ref, o_ref):
    xb = x_ref[...]
    ab = a_ref[0] + a_ref[1]
    dn = (((1,), (1,)), ((), ()))
    o_ref[...] = (
        lax.dot_general(xb, ws_ref[...], dn, preferred_element_type=jnp.float32)
        + lax.dot_general(ab, wn_ref[...], dn, preferred_element_type=jnp.float32)
        + b_ref[...]
    )


@functools.lru_cache(maxsize=None)
def _make_tc_linear(n_nodes: int, d: int):
    br = 1000
    assert n_nodes % br == 0
    grid = (n_nodes // br,)
    return pl.pallas_call(
        _tc_body,
        grid=grid,
        in_specs=[
            pl.BlockSpec((br, d), lambda i: (i, 0)),
            pl.BlockSpec((2, br, d), lambda i: (0, i, 0)),
            pl.BlockSpec((d, d), lambda i: (0, 0)),
            pl.BlockSpec((d, d), lambda i: (0, 0)),
            pl.BlockSpec((1, d), lambda i: (0, 0)),
        ],
        out_specs=pl.BlockSpec((br, d), lambda i: (i, 0)),
        out_shape=jax.ShapeDtypeStruct((n_nodes, d), jnp.float32),
    )


def kernel(x, edge_index, edge_weight, num_nodes, W_self, b_self, W_neigh, b_neigh):
    n, d = x.shape
    ei = edge_index.astype(jnp.int32)
    row = (ei[0] % jnp.asarray(num_nodes, jnp.int32)).astype(jnp.int32)
    col = ei[1]
    agg = _make_sc_aggregate(n, d, ei.shape[1])(
        row, col, edge_weight.astype(jnp.float32), x)
    bias = (b_self + b_neigh).reshape(1, d).astype(jnp.float32)
    return _make_tc_linear(n, d)(x, agg, W_self, W_neigh, bias)


# speedup vs baseline: 2.0374x; 1.0030x over previous
import functools

import jax
import jax.numpy as jnp
from jax import lax
from jax.experimental import pallas as pl
from jax.experimental.pallas import tpu as pltpu
from jax.experimental.pallas import tpu_sc as plsc

CH = 128
LANES = 16
NW = 32


@functools.lru_cache(maxsize=None)
def _make_sc_aggregate(n_nodes: int, d: int, n_edges: int):
    assert n_edges % CH == 0
    n_chunks = n_edges // CH
    per = n_chunks // NW
    per -= per % 3
    extra = n_chunks - per * NW

    rows_per_tile = (n_nodes // (16 * 8)) * 8
    hop = 104
    n_hops = rows_per_tile // hop
    assert n_hops * hop == rows_per_tile
    tail = n_nodes - 16 * rows_per_tile
    assert 0 <= tail <= CH and tail % 8 == 0
    vregs_per_row = d // LANES

    mesh = plsc.VectorSubcoreMesh(core_axis_name="c", subcore_axis_name="s")

    rows_t = pltpu.VMEM((CH, d), jnp.float32)
    idx_t = pltpu.VMEM((CH,), jnp.int32)
    w_t = pltpu.VMEM((CH,), jnp.float32)
    sem_t = pltpu.SemaphoreType.DMA

    @functools.partial(
        pl.kernel,
        mesh=mesh,
        out_type=jax.ShapeDtypeStruct((2, n_nodes, d), jnp.float32),
        scratch_types=[
            rows_t, rows_t, rows_t, idx_t, idx_t, idx_t, idx_t, idx_t, idx_t,
            w_t, w_t, w_t,
            pltpu.VMEM_SHARED((n_nodes, d), jnp.float32),
            sem_t, sem_t, sem_t, sem_t, sem_t, sem_t,
        ],
    )
    def sc_agg(row_hbm, col_hbm, w_hbm, x_hbm, out_hbm,
               rows0, rows1, rows2, col0, col1, col2, dst0, dst1, dst2,
               wgt0, wgt1, wgt2,
               accum, semg0, semg1, semg2, sems0, sems1, sems2):
        rows = (rows0, rows1, rows2)
        col = (col0, col1, col2)
        dst = (dst0, dst1, dst2)
        wgt = (wgt0, wgt1, wgt2)
        sem_g = (semg0, semg1, semg2)
        sem_s = (sems0, sems1, sems2)

        c = lax.axis_index("c")
        s = lax.axis_index("s")
        wid = s * 2 + c

        zero16 = jnp.zeros((LANES,), jnp.float32)

        def _zero_row(i, _):
            for j in range(vregs_per_row):
                rows0[i, pl.ds(j * LANES, LANES)] = zero16
            return 0

        lax.fori_loop(0, CH, _zero_row, 0)
        for h in range(n_hops):
            pltpu.sync_copy(rows0.at[pl.ds(0, hop)],
                            accum.at[pl.ds(s * rows_per_tile + h * hop, hop)])
        if tail:
            @pl.when(s == 15)
            def _():
                pltpu.sync_copy(rows0.at[pl.ds(0, tail)],
                                accum.at[pl.ds(16 * rows_per_tile, tail)])
        plsc.subcore_barrier()

        def _idx_load_colw(k, b):
            off = (k * NW + wid) * CH
            pltpu.sync_copy(col_hbm.at[pl.ds(off, CH)], col[b])
            pltpu.sync_copy(w_hbm.at[pl.ds(off, CH)], wgt[b])

        def _idx_load_dst(k, b):
            off = (k * NW + wid) * CH
            pltpu.sync_copy(row_hbm.at[pl.ds(off, CH)], dst[b])

        def _gather_start(b):
            pltpu.async_copy(x_hbm.at[col[b]], rows[b], sem_g[b])

        def _gather_wait(b):
            pltpu.make_async_copy(x_hbm.at[col[b]], rows[b], sem_g[b]).wait()

        def _scatter_start(b):
            pltpu.async_copy(rows[b], accum.at[dst[b]], sem_s[b], add=True)

        def _scatter_wait(b):
            pltpu.make_async_copy(rows[b], accum.at[dst[b]], sem_s[b]).wait()

        def _scale(b):
            def _group(g, _):
                w16 = wgt[b][pl.ds(g * LANES, LANES)]
                for lane in range(LANES):
                    e = g * LANES + lane
                    wvec = jnp.full((LANES,), w16[lane], jnp.float32)
                    for j in range(vregs_per_row):
                        rows[b][e, pl.ds(j * LANES, LANES)] = (
                            rows[b][e, pl.ds(j * LANES, LANES)] * wvec)
                return 0

            lax.fori_loop(0, CH // LANES, _group, 0)

        def _step(k, b):
            b2 = (b + 2) % 3

            @pl.when(k + 2 < per)
            def _():
                _idx_load_colw(k + 2, b2)

            _gather_wait(b)

            @pl.when(k + 2 < per)
            def _():
                @pl.when(k >= 1)
                def _():
                    _scatter_wait(b2)
                _gather_start(b2)
                _idx_load_dst(k + 2, b2)

            _scale(b)
            _scatter_start(b)

        for kk in range(2):
            _idx_load_colw(kk, kk)
            _idx_load_dst(kk, kk)
            _gather_start(kk)

        def _body(g, _):
            _step(3 * g, 0)
            _step(3 * g + 1, 1)
            _step(3 * g + 2, 2)
            return 0

        lax.fori_loop(0, per // 3, _body, 0)
        for bb in range(3):
            _scatter_wait(bb)

        if extra:
            @pl.when(wid < extra)
            def _():
                off = (per * NW + wid) * CH
                pltpu.sync_copy(col_hbm.at[pl.ds(off, CH)], col0)
                pltpu.sync_copy(row_hbm.at[pl.ds(off, CH)], dst0)
                pltpu.sync_copy(w_hbm.at[pl.ds(off, CH)], wgt0)
                _gather_start(0)
                _gather_wait(0)
                _scale(0)
                pltpu.sync_copy(rows0, accum.at[dst0], add=True)

        plsc.subcore_barrier()

        for h in range(n_hops):
            r0 = s * rows_per_tile + h * hop
            pltpu.sync_copy(accum.at[pl.ds(r0, hop)], rows0.at[pl.ds(0, hop)])
            pltpu.sync_copy(rows0.at[pl.ds(0, hop)], out_hbm.at[c, pl.ds(r0, hop)])
        if tail:
            @pl.when(s == 15)
            def _():
                r0 = 16 * rows_per_tile
                pltpu.sync_copy(accum.at[pl.ds(r0, tail)], rows0.at[pl.ds(0, tail)])
                pltpu.sync_copy(rows0.at[pl.ds(0, tail)],
                                out_hbm.at[c, pl.ds(r0, tail)])

    return sc_agg


def _tc_body(x_ref, a_ref, ws_ref, wn_ref, b_ref, o_ref):
    xb = x_ref[...]
    ab = a_ref[0] + a_ref[1]
    dn = (((1,), (1,)), ((), ()))
    o_ref[...] = (
        lax.dot_general(xb, ws_ref[...], dn, preferred_element_type=jnp.float32)
        + lax.dot_general(ab, wn_ref[...], dn, preferred_element_type=jnp.float32)
        + b_ref[...]
    )


@functools.lru_cache(maxsize=None)
def _make_tc_linear(n_nodes: int, d: int):
    br = 1000
    assert n_nodes % br == 0
    grid = (n_nodes // br,)
    return pl.pallas_call(
        _tc_body,
        grid=grid,
        in_specs=[
            pl.BlockSpec((br, d), lambda i: (i, 0)),
            pl.BlockSpec((2, br, d), lambda i: (0, i, 0)),
            pl.BlockSpec((d, d), lambda i: (0, 0)),
            pl.BlockSpec((d, d), lambda i: (0, 0)),
            pl.BlockSpec((1, d), lambda i: (0, 0)),
        ],
        out_specs=pl.BlockSpec((br, d), lambda i: (i, 0)),
        out_shape=jax.ShapeDtypeStruct((n_nodes, d), jnp.float32),
    )


def kernel(x, edge_index, edge_weight, num_nodes, W_self, b_self, W_neigh, b_neigh):
    n, d = x.shape
    ei = edge_index.astype(jnp.int32)
    row = (ei[0] % jnp.asarray(num_nodes, jnp.int32)).astype(jnp.int32)
    col = ei[1]
    agg = _make_sc_aggregate(n, d, ei.shape[1])(
        row, col, edge_weight.astype(jnp.float32), x)
    bias = (b_self + b_neigh).reshape(1, d).astype(jnp.float32)
    return _make_tc_linear(n, d)(x, agg, W_self, W_neigh, bias)


# speedup vs baseline: 2.2437x; 1.1012x over previous
import functools

import jax
import jax.numpy as jnp
from jax import lax
from jax.experimental import pallas as pl
from jax.experimental.pallas import tpu as pltpu
from jax.experimental.pallas import tpu_sc as plsc

CH = 128
LANES = 16
NW = 32


@functools.lru_cache(maxsize=None)
def _make_sc_aggregate(n_nodes: int, d: int, n_edges: int):
    assert n_edges % CH == 0
    n_chunks = n_edges // CH
    per = n_chunks // NW
    per -= per % 3
    extra = n_chunks - per * NW

    rows_per_tile = (n_nodes // (16 * 8)) * 8
    hop = 104
    n_hops = rows_per_tile // hop
    assert n_hops * hop == rows_per_tile
    tail = n_nodes - 16 * rows_per_tile
    assert 0 <= tail <= CH and tail % 8 == 0
    vregs_per_row = d // LANES

    mesh = plsc.VectorSubcoreMesh(core_axis_name="c", subcore_axis_name="s")

    rows_t = pltpu.VMEM((CH, d), jnp.float32)
    idx_t = pltpu.VMEM((CH,), jnp.int32)
    w_t = pltpu.VMEM((CH,), jnp.float32)
    sem_t = pltpu.SemaphoreType.DMA

    @functools.partial(
        pl.kernel,
        mesh=mesh,
        out_type=jax.ShapeDtypeStruct((2, n_nodes, d), jnp.float32),
        scratch_types=[
            rows_t, rows_t, rows_t, idx_t, idx_t, idx_t, idx_t, idx_t, idx_t,
            w_t, w_t, w_t,
            pltpu.VMEM_SHARED((n_nodes, d), jnp.float32),
            sem_t, sem_t, sem_t, sem_t, sem_t, sem_t,
        ],
    )
    def sc_agg(ei_hbm, w_hbm, x_hbm, out_hbm,
               rows0, rows1, rows2, col0, col1, col2, dst0, dst1, dst2,
               wgt0, wgt1, wgt2,
               accum, semg0, semg1, semg2, sems0, sems1, sems2):
        rows = (rows0, rows1, rows2)
        col = (col0, col1, col2)
        dst = (dst0, dst1, dst2)
        wgt = (wgt0, wgt1, wgt2)
        sem_g = (semg0, semg1, semg2)
        sem_s = (sems0, sems1, sems2)

        c = lax.axis_index("c")
        s = lax.axis_index("s")
        wid = s * 2 + c

        zero16 = jnp.zeros((LANES,), jnp.float32)

        def _zero_row(i, _):
            for j in range(vregs_per_row):
                rows0[i, pl.ds(j * LANES, LANES)] = zero16
            return 0

        lax.fori_loop(0, CH, _zero_row, 0)
        for h in range(n_hops):
            pltpu.sync_copy(rows0.at[pl.ds(0, hop)],
                            accum.at[pl.ds(s * rows_per_tile + h * hop, hop)])
        if tail:
            @pl.when(s == 15)
            def _():
                pltpu.sync_copy(rows0.at[pl.ds(0, tail)],
                                accum.at[pl.ds(16 * rows_per_tile, tail)])
        plsc.subcore_barrier()

        def _idx_load_colw(k, b):
            off = (k * NW + wid) * CH
            pltpu.sync_copy(ei_hbm.at[1, pl.ds(off, CH)], col[b])
            pltpu.sync_copy(w_hbm.at[pl.ds(off, CH)], wgt[b])

        def _idx_load_dst(k, b):
            off = (k * NW + wid) * CH
            pltpu.sync_copy(ei_hbm.at[0, pl.ds(off, CH)], dst[b])

        def _mod_dst(b):
            nn = jnp.full((LANES,), n_nodes, jnp.int32)
            for g in range(CH // LANES):
                dst[b][pl.ds(g * LANES, LANES)] = lax.rem(
                    dst[b][pl.ds(g * LANES, LANES)], nn)

        def _gather_start(b):
            pltpu.async_copy(x_hbm.at[col[b]], rows[b], sem_g[b])

        def _gather_wait(b):
            pltpu.make_async_copy(x_hbm.at[col[b]], rows[b], sem_g[b]).wait()

        def _scatter_start(b):
            pltpu.async_copy(rows[b], accum.at[dst[b]], sem_s[b], add=True)

        def _scatter_wait(b):
            pltpu.make_async_copy(rows[b], accum.at[dst[b]], sem_s[b]).wait()

        def _scale(b):
            def _group(g, _):
                w16 = wgt[b][pl.ds(g * LANES, LANES)]
                for lane in range(LANES):
                    e = g * LANES + lane
                    wvec = jnp.full((LANES,), w16[lane], jnp.float32)
                    for j in range(vregs_per_row):
                        rows[b][e, pl.ds(j * LANES, LANES)] = (
                            rows[b][e, pl.ds(j * LANES, LANES)] * wvec)
                return 0

            lax.fori_loop(0, CH // LANES, _group, 0)

        def _step(k, b):
            b2 = (b + 2) % 3

            @pl.when(k + 2 < per)
            def _():
                _idx_load_colw(k + 2, b2)

            _gather_wait(b)

            @pl.when(k + 2 < per)
            def _():
                @pl.when(k >= 1)
                def _():
                    _scatter_wait(b2)
                _gather_start(b2)
                _idx_load_dst(k + 2, b2)

            _mod_dst(b)
            _scale(b)
            _scatter_start(b)

        for kk in range(2):
            _idx_load_colw(kk, kk)
            _idx_load_dst(kk, kk)
            _gather_start(kk)

        def _body(g, _):
            _step(3 * g, 0)
            _step(3 * g + 1, 1)
            _step(3 * g + 2, 2)
            return 0

        lax.fori_loop(0, per // 3, _body, 0)
        for bb in range(3):
            _scatter_wait(bb)

        if extra:
            @pl.when(wid < extra)
            def _():
                off = (per * NW + wid) * CH
                pltpu.sync_copy(ei_hbm.at[1, pl.ds(off, CH)], col0)
                pltpu.sync_copy(ei_hbm.at[0, pl.ds(off, CH)], dst0)
                pltpu.sync_copy(w_hbm.at[pl.ds(off, CH)], wgt0)
                _gather_start(0)
                _gather_wait(0)
                _mod_dst(0)
                _scale(0)
                pltpu.sync_copy(rows0, accum.at[dst0], add=True)

        plsc.subcore_barrier()

        for h in range(n_hops):
            r0 = s * rows_per_tile + h * hop
            pltpu.sync_copy(accum.at[pl.ds(r0, hop)], rows0.at[pl.ds(0, hop)])
            pltpu.sync_copy(rows0.at[pl.ds(0, hop)], out_hbm.at[c, pl.ds(r0, hop)])
        if tail:
            @pl.when(s == 15)
            def _():
                r0 = 16 * rows_per_tile
                pltpu.sync_copy(accum.at[pl.ds(r0, tail)], rows0.at[pl.ds(0, tail)])
                pltpu.sync_copy(rows0.at[pl.ds(0, tail)],
                                out_hbm.at[c, pl.ds(r0, tail)])

    return sc_agg


def _tc_body(x_ref, a_ref, ws_ref, wn_ref, b_ref, o_ref):
    xb = x_ref[...]
    ab = a_ref[0] + a_ref[1]
    dn = (((1,), (1,)), ((), ()))
    o_ref[...] = (
        lax.dot_general(xb, ws_ref[...], dn, preferred_element_type=jnp.float32)
        + lax.dot_general(ab, wn_ref[...], dn, preferred_element_type=jnp.float32)
        + b_ref[...]
    )


@functools.lru_cache(maxsize=None)
def _make_tc_linear(n_nodes: int, d: int):
    br = 1000
    assert n_nodes % br == 0
    grid = (n_nodes // br,)
    return pl.pallas_call(
        _tc_body,
        grid=grid,
        in_specs=[
            pl.BlockSpec((br, d), lambda i: (i, 0)),
            pl.BlockSpec((2, br, d), lambda i: (0, i, 0)),
            pl.BlockSpec((d, d), lambda i: (0, 0)),
            pl.BlockSpec((d, d), lambda i: (0, 0)),
            pl.BlockSpec((1, d), lambda i: (0, 0)),
        ],
        out_specs=pl.BlockSpec((br, d), lambda i: (i, 0)),
        out_shape=jax.ShapeDtypeStruct((n_nodes, d), jnp.float32),
    )


def kernel(x, edge_index, edge_weight, num_nodes, W_self, b_self, W_neigh, b_neigh):
    n, d = x.shape
    ei = edge_index.astype(jnp.int32)
    agg = _make_sc_aggregate(n, d, ei.shape[1])(
        ei, edge_weight.astype(jnp.float32), x)
    bias = (b_self + b_neigh).reshape(1, d).astype(jnp.float32)
    return _make_tc_linear(n, d)(x, agg, W_self, W_neigh, bias)


# speedup vs baseline: 3.3112x; 1.4758x over previous
import functools

import jax
import jax.numpy as jnp
from jax import lax
from jax.experimental import pallas as pl
from jax.experimental.pallas import tpu as pltpu
from jax.experimental.pallas import tpu_sc as plsc

CH = 128
LANES = 16
NW = 32
PH = 26
NPH = 3


@functools.lru_cache(maxsize=None)
def _make_sc_aggregate(n_nodes: int, d: int, n_edges: int):
    assert n_edges % CH == 0
    n_chunks = n_edges // CH
    per = PH * NPH
    assert per <= n_chunks // NW
    extra = n_chunks - per * NW
    assert extra <= NW

    rows_per_tile = (n_nodes // (16 * 8)) * 8
    hop = 104
    n_hops = rows_per_tile // hop
    assert n_hops * hop == rows_per_tile
    tail = n_nodes - 16 * rows_per_tile
    assert 0 <= tail <= CH and tail % 8 == 0
    vregs_per_row = d // LANES

    mesh = plsc.VectorSubcoreMesh(core_axis_name="c", subcore_axis_name="s")

    rows_t = pltpu.VMEM((CH, d), jnp.float32)
    idx_t = pltpu.VMEM((CH,), jnp.int32)
    w_t = pltpu.VMEM((CH,), jnp.float32)
    sidx_t = pltpu.VMEM((PH * CH,), jnp.int32)
    sw_t = pltpu.VMEM((PH * CH,), jnp.float32)
    sem_t = pltpu.SemaphoreType.DMA

    @functools.partial(
        pl.kernel,
        mesh=mesh,
        out_type=jax.ShapeDtypeStruct((2, n_nodes, d), jnp.float32),
        scratch_types=[
            rows_t, rows_t, idx_t, idx_t, idx_t, idx_t, w_t, w_t,
            sidx_t, sidx_t, sw_t,
            pltpu.VMEM_SHARED((n_nodes, d), jnp.float32),
            sem_t, sem_t, sem_t, sem_t,
        ],
    )
    def sc_agg(ei_hbm, w_hbm, x_hbm, out_hbm,
               rows0, rows1, col0, col1, dst0, dst1, wgt0, wgt1,
               col_s, dst_s, wgt_s,
               accum, semg0, semg1, sems0, sems1):
        rows = (rows0, rows1)
        col = (col0, col1)
        dst = (dst0, dst1)
        wgt = (wgt0, wgt1)
        sem_g = (semg0, semg1)
        sem_s = (sems0, sems1)

        c = lax.axis_index("c")
        s = lax.axis_index("s")
        wid = s * 2 + c
        nn = jnp.full((LANES,), n_nodes, jnp.int32)

        zero16 = jnp.zeros((LANES,), jnp.float32)

        def _zero_row(i, _):
            for j in range(vregs_per_row):
                rows0[i, pl.ds(j * LANES, LANES)] = zero16
            return 0

        lax.fori_loop(0, CH, _zero_row, 0)
        for h in range(n_hops):
            pltpu.sync_copy(rows0.at[pl.ds(0, hop)],
                            accum.at[pl.ds(s * rows_per_tile + h * hop, hop)])
        if tail:
            @pl.when(s == 15)
            def _():
                pltpu.sync_copy(rows0.at[pl.ds(0, tail)],
                                accum.at[pl.ds(16 * rows_per_tile, tail)])
        plsc.subcore_barrier()

        def _prep_colw(l, b):
            def _cp(j, _):
                col[b][pl.ds(j * LANES, LANES)] = (
                    col_s[pl.ds(l * CH + j * LANES, LANES)])
                wgt[b][pl.ds(j * LANES, LANES)] = (
                    wgt_s[pl.ds(l * CH + j * LANES, LANES)])
                return 0

            lax.fori_loop(0, CH // LANES, _cp, 0)

        def _prep_dst(l, b):
            def _cp(j, _):
                dst[b][pl.ds(j * LANES, LANES)] = lax.rem(
                    dst_s[pl.ds(l * CH + j * LANES, LANES)], nn)
                return 0

            lax.fori_loop(0, CH // LANES, _cp, 0)

        def _gather_start(b):
            pltpu.async_copy(x_hbm.at[col[b]], rows[b], sem_g[b])

        def _gather_wait(b):
            pltpu.make_async_copy(x_hbm.at[col[b]], rows[b], sem_g[b]).wait()

        def _scatter_start(b):
            pltpu.async_copy(rows[b], accum.at[dst[b]], sem_s[b], add=True)

        def _scatter_wait(b):
            pltpu.make_async_copy(rows[b], accum.at[dst[b]], sem_s[b]).wait()

        def _scale(b):
            def _group(g, _):
                w16 = wgt[b][pl.ds(g * LANES, LANES)]
                for lane in range(LANES):
                    e = g * LANES + lane
                    wvec = jnp.full((LANES,), w16[lane], jnp.float32)
                    for j in range(vregs_per_row):
                        rows[b][e, pl.ds(j * LANES, LANES)] = (
                            rows[b][e, pl.ds(j * LANES, LANES)] * wvec)
                return 0

            lax.fori_loop(0, CH // LANES, _group, 0)

        def _step(l, b):
            bn = (b + 1) % 2

            @pl.when(l + 1 < PH)
            def _():
                _prep_colw(l + 1, bn)

            _gather_wait(b)

            @pl.when(l >= 1)
            def _():
                _scatter_wait(bn)

            @pl.when(l + 1 < PH)
            def _():
                _gather_start(bn)
                _prep_dst(l + 1, bn)

            _scale(b)
            _scatter_start(b)

        for p in range(NPH):
            off = (wid * per + p * PH) * CH
            pltpu.sync_copy(ei_hbm.at[1, pl.ds(off, PH * CH)], col_s)
            pltpu.sync_copy(ei_hbm.at[0, pl.ds(off, PH * CH)], dst_s)
            pltpu.sync_copy(w_hbm.at[pl.ds(off, PH * CH)], wgt_s)
            _prep_colw(0, 0)
            _prep_dst(0, 0)
            _gather_start(0)

            def _body(g, _):
                _step(2 * g, 0)
                _step(2 * g + 1, 1)
                return 0

            lax.fori_loop(0, PH // 2, _body, 0)
            _scatter_wait((PH - 1) % 2)

        if extra:
            @pl.when(wid < extra)
            def _():
                off = (per * NW + wid) * CH
                pltpu.sync_copy(ei_hbm.at[1, pl.ds(off, CH)], col0)
                pltpu.sync_copy(ei_hbm.at[0, pl.ds(off, CH)], dst0)
                pltpu.sync_copy(w_hbm.at[pl.ds(off, CH)], wgt0)
                _gather_start(0)
                _gather_wait(0)
                def _md(j, _):
                    dst0[pl.ds(j * LANES, LANES)] = lax.rem(
                        dst0[pl.ds(j * LANES, LANES)], nn)
                    return 0
                lax.fori_loop(0, CH // LANES, _md, 0)
                _scale(0)
                pltpu.sync_copy(rows0, accum.at[dst0], add=True)

        plsc.subcore_barrier()

        for h in range(n_hops):
            r0 = s * rows_per_tile + h * hop
            pltpu.sync_copy(accum.at[pl.ds(r0, hop)], rows0.at[pl.ds(0, hop)])
            pltpu.sync_copy(rows0.at[pl.ds(0, hop)], out_hbm.at[c, pl.ds(r0, hop)])
        if tail:
            @pl.when(s == 15)
            def _():
                r0 = 16 * rows_per_tile
                pltpu.sync_copy(accum.at[pl.ds(r0, tail)], rows0.at[pl.ds(0, tail)])
                pltpu.sync_copy(rows0.at[pl.ds(0, tail)],
                                out_hbm.at[c, pl.ds(r0, tail)])

    return sc_agg


def _tc_body(x_ref, a_ref, ws_ref, wn_ref, b_ref, o_ref):
    xb = x_ref[...]
    ab = a_ref[0] + a_ref[1]
    dn = (((1,), (1,)), ((), ()))
    o_ref[...] = (
        lax.dot_general(xb, ws_ref[...], dn, preferred_element_type=jnp.float32)
        + lax.dot_general(ab, wn_ref[...], dn, preferred_element_type=jnp.float32)
        + b_ref[...]
    )


@functools.lru_cache(maxsize=None)
def _make_tc_linear(n_nodes: int, d: int):
    br = 1000
    assert n_nodes % br == 0
    grid = (n_nodes // br,)
    return pl.pallas_call(
        _tc_body,
        grid=grid,
        in_specs=[
            pl.BlockSpec((br, d), lambda i: (i, 0)),
            pl.BlockSpec((2, br, d), lambda i: (0, i, 0)),
            pl.BlockSpec((d, d), lambda i: (0, 0)),
            pl.BlockSpec((d, d), lambda i: (0, 0)),
            pl.BlockSpec((1, d), lambda i: (0, 0)),
        ],
        out_specs=pl.BlockSpec((br, d), lambda i: (i, 0)),
        out_shape=jax.ShapeDtypeStruct((n_nodes, d), jnp.float32),
    )


def kernel(x, edge_index, edge_weight, num_nodes, W_self, b_self, W_neigh, b_neigh):
    n, d = x.shape
    ei = edge_index.astype(jnp.int32)
    agg = _make_sc_aggregate(n, d, ei.shape[1])(
        ei, edge_weight.astype(jnp.float32), x)
    bias = (b_self + b_neigh).reshape(1, d).astype(jnp.float32)
    return _make_tc_linear(n, d)(x, agg, W_self, W_neigh, bias)


# speedup vs baseline: 3.3852x; 1.0223x over previous
import functools

import jax
import jax.numpy as jnp
from jax import lax
from jax.experimental import pallas as pl
from jax.experimental.pallas import tpu as pltpu
from jax.experimental.pallas import tpu_sc as plsc

CH = 128
LANES = 16
NW = 32
PH = 26
NPH = 3


@functools.lru_cache(maxsize=None)
def _make_sc_aggregate(n_nodes: int, d: int, n_edges: int):
    assert n_edges % CH == 0
    n_chunks = n_edges // CH
    per = PH * NPH
    assert per <= n_chunks // NW
    extra = n_chunks - per * NW
    assert extra <= NW

    rows_per_tile = (n_nodes // (16 * 8)) * 8
    hop = 104
    n_hops = rows_per_tile // hop
    assert n_hops * hop == rows_per_tile
    tail = n_nodes - 16 * rows_per_tile
    assert 0 <= tail <= CH and tail % 8 == 0
    vregs_per_row = d // LANES

    mesh = plsc.VectorSubcoreMesh(core_axis_name="c", subcore_axis_name="s")

    rows_t = pltpu.VMEM((CH, d), jnp.float32)
    idx_t = pltpu.VMEM((CH,), jnp.int32)
    w_t = pltpu.VMEM((CH,), jnp.float32)
    sidx_t = pltpu.VMEM((PH * CH,), jnp.int32)
    sw_t = pltpu.VMEM((PH * CH,), jnp.float32)
    sem_t = pltpu.SemaphoreType.DMA

    @functools.partial(
        pl.kernel,
        mesh=mesh,
        out_type=jax.ShapeDtypeStruct((2, n_nodes, d), jnp.float32),
        scratch_types=[
            rows_t, rows_t, idx_t, idx_t, idx_t, idx_t, w_t, w_t,
            sidx_t, sidx_t, sw_t,
            pltpu.VMEM_SHARED((n_nodes, d), jnp.float32),
            sem_t, sem_t, sem_t, sem_t,
        ],
    )
    def sc_agg(ei_hbm, w_hbm, x_hbm, out_hbm,
               rows0, rows1, col0, col1, dst0, dst1, wgt0, wgt1,
               col_s, dst_s, wgt_s,
               accum, semg0, semg1, sems0, sems1):
        rows = (rows0, rows1)
        col = (col0, col1)
        dst = (dst0, dst1)
        wgt = (wgt0, wgt1)
        sem_g = (semg0, semg1)
        sem_s = (sems0, sems1)

        c = lax.axis_index("c")
        s = lax.axis_index("s")
        wid = s * 2 + c
        nn = jnp.full((LANES,), n_nodes, jnp.int32)

        zero16 = jnp.zeros((LANES,), jnp.float32)

        def _zero_row(i, _):
            for j in range(vregs_per_row):
                rows0[i, pl.ds(j * LANES, LANES)] = zero16
            return 0

        lax.fori_loop(0, CH, _zero_row, 0)
        for h in range(n_hops):
            pltpu.sync_copy(rows0.at[pl.ds(0, hop)],
                            accum.at[pl.ds(s * rows_per_tile + h * hop, hop)])
        if tail:
            @pl.when(s == 15)
            def _():
                pltpu.sync_copy(rows0.at[pl.ds(0, tail)],
                                accum.at[pl.ds(16 * rows_per_tile, tail)])
        plsc.subcore_barrier()

        def _prep_colw(l, b):
            def _cp(j, _):
                col[b][pl.ds(j * LANES, LANES)] = (
                    col_s[pl.ds(l * CH + j * LANES, LANES)])
                wgt[b][pl.ds(j * LANES, LANES)] = (
                    wgt_s[pl.ds(l * CH + j * LANES, LANES)])
                return 0

            lax.fori_loop(0, CH // LANES, _cp, 0)

        def _prep_dst(l, b):
            def _cp(j, _):
                dst[b][pl.ds(j * LANES, LANES)] = lax.rem(
                    dst_s[pl.ds(l * CH + j * LANES, LANES)], nn)
                return 0

            lax.fori_loop(0, CH // LANES, _cp, 0)

        def _gather_start(b):
            pltpu.async_copy(x_hbm.at[col[b]], rows[b], sem_g[b])

        def _gather_wait(b):
            pltpu.make_async_copy(x_hbm.at[col[b]], rows[b], sem_g[b]).wait()

        def _scatter_start(b):
            pltpu.async_copy(rows[b], accum.at[dst[b]], sem_s[b], add=True)

        def _scatter_wait(b):
            pltpu.make_async_copy(rows[b], accum.at[dst[b]], sem_s[b]).wait()

        def _scale(b):
            def _group(g, _):
                w16 = wgt[b][pl.ds(g * LANES, LANES)]
                for lane in range(LANES):
                    e = g * LANES + lane
                    wvec = jnp.full((LANES,), w16[lane], jnp.float32)
                    for j in range(vregs_per_row):
                        rows[b][e, pl.ds(j * LANES, LANES)] = (
                            rows[b][e, pl.ds(j * LANES, LANES)] * wvec)
                return 0

            lax.fori_loop(0, CH // LANES, _group, 0)

        def _step(l, b):
            bn = (b + 1) % 2

            @pl.when(l + 1 < PH)
            def _():
                _prep_colw(l + 1, bn)

            _gather_wait(b)

            @pl.when(l >= 1)
            def _():
                _scatter_wait(bn)

            @pl.when(l + 1 < PH)
            def _():
                _gather_start(bn)
                _prep_dst(l + 1, bn)

            _scale(b)
            _scatter_start(b)

        for p in range(NPH):
            off = (wid * per + p * PH) * CH
            pltpu.async_copy(ei_hbm.at[1, pl.ds(off, PH * CH)], col_s, semg0)
            pltpu.async_copy(ei_hbm.at[0, pl.ds(off, PH * CH)], dst_s, semg0)
            pltpu.async_copy(w_hbm.at[pl.ds(off, PH * CH)], wgt_s, semg0)
            pltpu.make_async_copy(ei_hbm.at[1, pl.ds(off, PH * CH)], col_s, semg0).wait()
            pltpu.make_async_copy(ei_hbm.at[0, pl.ds(off, PH * CH)], dst_s, semg0).wait()
            pltpu.make_async_copy(w_hbm.at[pl.ds(off, PH * CH)], wgt_s, semg0).wait()
            _prep_colw(0, 0)
            _prep_dst(0, 0)
            _gather_start(0)

            def _body(g, _):
                _step(2 * g, 0)
                _step(2 * g + 1, 1)
                return 0

            lax.fori_loop(0, PH // 2, _body, 0)
            _scatter_wait((PH - 1) % 2)

        if extra:
            @pl.when(wid < extra)
            def _():
                off = (per * NW + wid) * CH
                pltpu.sync_copy(ei_hbm.at[1, pl.ds(off, CH)], col0)
                pltpu.sync_copy(ei_hbm.at[0, pl.ds(off, CH)], dst0)
                pltpu.sync_copy(w_hbm.at[pl.ds(off, CH)], wgt0)
                _gather_start(0)
                _gather_wait(0)
                def _md(j, _):
                    dst0[pl.ds(j * LANES, LANES)] = lax.rem(
                        dst0[pl.ds(j * LANES, LANES)], nn)
                    return 0
                lax.fori_loop(0, CH // LANES, _md, 0)
                _scale(0)
                pltpu.sync_copy(rows0, accum.at[dst0], add=True)

        plsc.subcore_barrier()

        def _rd(h):
            r0 = s * rows_per_tile + h * hop
            return (accum.at[pl.ds(r0, hop)], rows[h % 2].at[pl.ds(0, hop)],
                    sem_g[h % 2])

        def _wr(h):
            r0 = s * rows_per_tile + h * hop
            return (rows[h % 2].at[pl.ds(0, hop)],
                    out_hbm.at[c, pl.ds(r0, hop)], sem_s[h % 2])

        pltpu.async_copy(*_rd(0))
        for h in range(n_hops):
            pltpu.make_async_copy(*_rd(h)).wait()
            pltpu.async_copy(*_wr(h))
            if h + 1 < n_hops:
                if h >= 1:
                    pltpu.make_async_copy(*_wr(h - 1)).wait()
                pltpu.async_copy(*_rd(h + 1))
        for h in (n_hops - 2, n_hops - 1):
            pltpu.make_async_copy(*_wr(h)).wait()
        if tail:
            @pl.when(s == 15)
            def _():
                r0 = 16 * rows_per_tile
                pltpu.sync_copy(accum.at[pl.ds(r0, tail)], rows0.at[pl.ds(0, tail)])
                pltpu.sync_copy(rows0.at[pl.ds(0, tail)],
                                out_hbm.at[c, pl.ds(r0, tail)])

    return sc_agg


def _tc_body(x_ref, a_ref, ws_ref, wn_ref, b_ref, o_ref):
    xb = x_ref[...]
    ab = a_ref[0] + a_ref[1]
    dn = (((1,), (1,)), ((), ()))
    o_ref[...] = (
        lax.dot_general(xb, ws_ref[...], dn, preferred_element_type=jnp.float32)
        + lax.dot_general(ab, wn_ref[...], dn, preferred_element_type=jnp.float32)
        + b_ref[...]
    )


@functools.lru_cache(maxsize=None)
def _make_tc_linear(n_nodes: int, d: int):
    br = 1000
    assert n_nodes % br == 0
    grid = (n_nodes // br,)
    return pl.pallas_call(
        _tc_body,
        grid=grid,
        in_specs=[
            pl.BlockSpec((br, d), lambda i: (i, 0)),
            pl.BlockSpec((2, br, d), lambda i: (0, i, 0)),
            pl.BlockSpec((d, d), lambda i: (0, 0)),
            pl.BlockSpec((d, d), lambda i: (0, 0)),
            pl.BlockSpec((1, d), lambda i: (0, 0)),
        ],
        out_specs=pl.BlockSpec((br, d), lambda i: (i, 0)),
        out_shape=jax.ShapeDtypeStruct((n_nodes, d), jnp.float32),
    )


def kernel(x, edge_index, edge_weight, num_nodes, W_self, b_self, W_neigh, b_neigh):
    n, d = x.shape
    ei = edge_index.astype(jnp.int32)
    agg = _make_sc_aggregate(n, d, ei.shape[1])(
        ei, edge_weight.astype(jnp.float32), x)
    bias = (b_self + b_neigh).reshape(1, d).astype(jnp.float32)
    return _make_tc_linear(n, d)(x, agg, W_self, W_neigh, bias)


# speedup vs baseline: 3.4582x; 1.0216x over previous
import functools

import jax
import jax.numpy as jnp
from jax import lax
from jax.experimental import pallas as pl
from jax.experimental.pallas import tpu as pltpu
from jax.experimental.pallas import tpu_sc as plsc

CH = 128
LANES = 16
NW = 32
PH = 26
NPH = 3


@functools.lru_cache(maxsize=None)
def _make_sc_aggregate(n_nodes: int, d: int, n_edges: int):
    assert n_edges % CH == 0
    n_chunks = n_edges // CH
    per = PH * NPH
    assert per <= n_chunks // NW
    extra = n_chunks - per * NW
    assert extra <= NW

    rows_per_tile = (n_nodes // (16 * 8)) * 8
    hop = 104
    n_hops = rows_per_tile // hop
    assert n_hops * hop == rows_per_tile
    tail = n_nodes - 16 * rows_per_tile
    assert 0 <= tail <= CH and tail % 8 == 0
    vregs_per_row = d // LANES

    mesh = plsc.VectorSubcoreMesh(core_axis_name="c", subcore_axis_name="s")

    rows_t = pltpu.VMEM((CH, d), jnp.float32)
    idx_t = pltpu.VMEM((CH,), jnp.int32)
    w_t = pltpu.VMEM((CH,), jnp.float32)
    sidx_t = pltpu.VMEM((PH * CH,), jnp.int32)
    sw_t = pltpu.VMEM((PH * CH,), jnp.float32)
    sem_t = pltpu.SemaphoreType.DMA

    @functools.partial(
        pl.kernel,
        mesh=mesh,
        out_type=jax.ShapeDtypeStruct((2, n_nodes, d), jnp.float32),
        scratch_types=[
            rows_t, rows_t, idx_t, idx_t, idx_t, idx_t, w_t, w_t,
            sidx_t, sidx_t, sw_t,
            pltpu.VMEM_SHARED((n_nodes, d), jnp.float32),
            sem_t, sem_t, sem_t, sem_t,
        ],
    )
    def sc_agg(ei_hbm, w_hbm, x_hbm, out_hbm,
               rows0, rows1, col0, col1, dst0, dst1, wgt0, wgt1,
               col_s, dst_s, wgt_s,
               accum, semg0, semg1, sems0, sems1):
        rows = (rows0, rows1)
        col = (col0, col1)
        dst = (dst0, dst1)
        wgt = (wgt0, wgt1)
        sem_g = (semg0, semg1)
        sem_s = (sems0, sems1)

        c = lax.axis_index("c")
        s = lax.axis_index("s")
        wid = s * 2 + c
        nn = jnp.full((LANES,), n_nodes, jnp.int32)

        zero16 = jnp.zeros((LANES,), jnp.float32)

        def _zero_row(i, _):
            for j in range(vregs_per_row):
                rows0[i, pl.ds(j * LANES, LANES)] = zero16
            return 0

        lax.fori_loop(0, CH, _zero_row, 0)
        for h in range(n_hops):
            pltpu.sync_copy(rows0.at[pl.ds(0, hop)],
                            accum.at[pl.ds(s * rows_per_tile + h * hop, hop)])
        if tail:
            @pl.when(s == 15)
            def _():
                pltpu.sync_copy(rows0.at[pl.ds(0, tail)],
                                accum.at[pl.ds(16 * rows_per_tile, tail)])
        plsc.subcore_barrier()

        def _prep_colw(l, b):
            def _cp(j, _):
                col[b][pl.ds(j * LANES, LANES)] = (
                    col_s[pl.ds(l * CH + j * LANES, LANES)])
                wgt[b][pl.ds(j * LANES, LANES)] = (
                    wgt_s[pl.ds(l * CH + j * LANES, LANES)])
                return 0

            lax.fori_loop(0, CH // LANES, _cp, 0)

        def _prep_dst(l, b):
            def _cp(j, _):
                dst[b][pl.ds(j * LANES, LANES)] = lax.rem(
                    dst_s[pl.ds(l * CH + j * LANES, LANES)], nn)
                return 0

            lax.fori_loop(0, CH // LANES, _cp, 0)

        def _gather_start(b):
            pltpu.async_copy(x_hbm.at[col[b]], rows[b], sem_g[b])

        def _gather_wait(b):
            pltpu.make_async_copy(x_hbm.at[col[b]], rows[b], sem_g[b]).wait()

        def _scatter_start(b):
            pltpu.async_copy(rows[b], accum.at[dst[b]], sem_s[b], add=True)

        def _scatter_wait(b):
            pltpu.make_async_copy(rows[b], accum.at[dst[b]], sem_s[b]).wait()

        def _scale(b):
            def _group(g, _):
                w16 = wgt[b][pl.ds(g * LANES, LANES)]
                for lane in range(LANES):
                    e = g * LANES + lane
                    wvec = jnp.full((LANES,), w16[lane], jnp.float32)
                    for j in range(vregs_per_row):
                        rows[b][e, pl.ds(j * LANES, LANES)] = (
                            rows[b][e, pl.ds(j * LANES, LANES)] * wvec)
                return 0

            lax.fori_loop(0, CH // LANES, _group, 0)

        def _step(l, b):
            bn = (b + 1) % 2

            @pl.when(l + 1 < PH)
            def _():
                _prep_colw(l + 1, bn)

            _gather_wait(b)

            @pl.when(l >= 1)
            def _():
                _scatter_wait(bn)

            @pl.when(l + 1 < PH)
            def _():
                _gather_start(bn)
                _prep_dst(l + 1, bn)

            _scale(b)
            _scatter_start(b)

        for p in range(NPH):
            off = (wid * per + p * PH) * CH
            pltpu.async_copy(ei_hbm.at[1, pl.ds(off, PH * CH)], col_s, semg0)
            pltpu.async_copy(ei_hbm.at[0, pl.ds(off, PH * CH)], dst_s, semg0)
            pltpu.async_copy(w_hbm.at[pl.ds(off, PH * CH)], wgt_s, semg0)
            pltpu.make_async_copy(ei_hbm.at[1, pl.ds(off, PH * CH)], col_s, semg0).wait()
            pltpu.make_async_copy(ei_hbm.at[0, pl.ds(off, PH * CH)], dst_s, semg0).wait()
            pltpu.make_async_copy(w_hbm.at[pl.ds(off, PH * CH)], wgt_s, semg0).wait()
            _prep_colw(0, 0)
            _prep_dst(0, 0)
            _gather_start(0)

            def _body(g, _):
                _step(2 * g, 0)
                _step(2 * g + 1, 1)
                return 0

            lax.fori_loop(0, PH // 2, _body, 0)
            _scatter_wait((PH - 1) % 2)

        if extra:
            @pl.when(wid < extra)
            def _():
                off = (per * NW + wid) * CH
                pltpu.sync_copy(ei_hbm.at[1, pl.ds(off, CH)], col0)
                pltpu.sync_copy(ei_hbm.at[0, pl.ds(off, CH)], dst0)
                pltpu.sync_copy(w_hbm.at[pl.ds(off, CH)], wgt0)
                _gather_start(0)
                _gather_wait(0)
                def _md(j, _):
                    dst0[pl.ds(j * LANES, LANES)] = lax.rem(
                        dst0[pl.ds(j * LANES, LANES)], nn)
                    return 0
                lax.fori_loop(0, CH // LANES, _md, 0)
                _scale(0)
                pltpu.sync_copy(rows0, accum.at[dst0], add=True)

        plsc.subcore_barrier()

        def _rd(h):
            r0 = s * rows_per_tile + h * hop
            return (accum.at[pl.ds(r0, hop)], rows[h % 2].at[pl.ds(0, hop)],
                    sem_g[h % 2])

        def _wr(h):
            r0 = s * rows_per_tile + h * hop
            return (rows[h % 2].at[pl.ds(0, hop)],
                    out_hbm.at[c, pl.ds(r0, hop)], sem_s[h % 2])

        pltpu.async_copy(*_rd(0))
        for h in range(n_hops):
            pltpu.make_async_copy(*_rd(h)).wait()
            pltpu.async_copy(*_wr(h))
            if h + 1 < n_hops:
                if h >= 1:
                    pltpu.make_async_copy(*_wr(h - 1)).wait()
                pltpu.async_copy(*_rd(h + 1))
        for h in (n_hops - 2, n_hops - 1):
            pltpu.make_async_copy(*_wr(h)).wait()
        if tail:
            @pl.when(s == 15)
            def _():
                r0 = 16 * rows_per_tile
                pltpu.sync_copy(accum.at[pl.ds(r0, tail)], rows0.at[pl.ds(0, tail)])
                pltpu.sync_copy(rows0.at[pl.ds(0, tail)],
                                out_hbm.at[c, pl.ds(r0, tail)])

    return sc_agg


def _tc_body(x_ref, a_ref, ws_ref, wn_ref, b_ref, o_ref):
    xb = x_ref[...]
    ab = a_ref[0] + a_ref[1]
    dn = (((1,), (1,)), ((), ()))
    o_ref[...] = (
        lax.dot_general(xb, ws_ref[...], dn, preferred_element_type=jnp.float32)
        + lax.dot_general(ab, wn_ref[...], dn, preferred_element_type=jnp.float32)
        + b_ref[...]
    )


@functools.lru_cache(maxsize=None)
def _make_tc_linear(n_nodes: int, d: int):
    br = 2000
    assert n_nodes % br == 0
    grid = (n_nodes // br,)
    return pl.pallas_call(
        _tc_body,
        grid=grid,
        in_specs=[
            pl.BlockSpec((br, d), lambda i: (i, 0)),
            pl.BlockSpec((2, br, d), lambda i: (0, i, 0)),
            pl.BlockSpec((d, d), lambda i: (0, 0)),
            pl.BlockSpec((d, d), lambda i: (0, 0)),
            pl.BlockSpec((1, d), lambda i: (0, 0)),
        ],
        out_specs=pl.BlockSpec((br, d), lambda i: (i, 0)),
        out_shape=jax.ShapeDtypeStruct((n_nodes, d), jnp.float32),
    )


def kernel(x, edge_index, edge_weight, num_nodes, W_self, b_self, W_neigh, b_neigh):
    n, d = x.shape
    ei = edge_index.astype(jnp.int32)
    agg = _make_sc_aggregate(n, d, ei.shape[1])(
        ei, edge_weight.astype(jnp.float32), x)
    bias = (b_self + b_neigh).reshape(1, d).astype(jnp.float32)
    return _make_tc_linear(n, d)(x, agg, W_self, W_neigh, bias)
